# baseline (speedup 1.0000x reference)
import functools

import jax
import jax.numpy as jnp
from jax import lax
from jax.experimental import pallas as pl
from jax.experimental.pallas import tpu as pltpu
from jax.experimental.pallas import tpu_sc as plsc

B = 1024
Q = 16
ROW = Q * 128 * 2
NW = 32
BPW = B // NW
L = 16
RPB = ROW // 128
NSLOT = 4
CHUNK = 4
NCH = BPW // CHUNK

_PIB = lax.GatherScatterMode.PROMISE_IN_BOUNDS
_DNUMS = lax.GatherDimensionNumbers(
    offset_dims=(), collapsed_slice_dims=(0,), start_index_map=(0,))


def _permute(x, idx):
    return lax.gather(x, idx[:, None], _DNUMS, slice_sizes=(1,), mode=_PIB)


def _trig_body(w_ref, ab_ref):
    th = w_ref[...] * 0.5
    a = jnp.cos(th)
    b = jnp.sin(th)
    for w in range(NW):
        ab_ref[w, 0:4, :] = a[4 * w:4 * w + 4, :]
        ab_ref[w, 4:8, :] = b[4 * w:4 * w + 4, :]


def _trig(w2):
    return pl.pallas_call(
        _trig_body,
        out_shape=jax.ShapeDtypeStruct((NW, 8, 128), jnp.float32),
    )(w2)


def _sc_rotate(state2d, ab):
    mesh = plsc.VectorSubcoreMesh(core_axis_name="c", subcore_axis_name="s")

    @functools.partial(
        pl.kernel,
        mesh=mesh,
        out_type=jax.ShapeDtypeStruct((B * RPB, 128), jnp.float32),
        scratch_types=[
            pltpu.VMEM((NSLOT, CHUNK * RPB, 128), jnp.float32),
            pltpu.VMEM((8, 128), jnp.float32),
        ] + [pltpu.SemaphoreType.DMA] * (2 * NSLOT),
        compiler_params=pltpu.CompilerParams(use_tc_tiling_on_sc=True),
    )
    def body(state_hbm, ab_hbm, out_hbm, buf, abv, *sems):
        sin_, sout = sems[:NSLOT], sems[NSLOT:]
        wid = lax.axis_index("s") * 2 + lax.axis_index("c")
        bb = wid * BPW
        pltpu.sync_copy(ab_hbm.at[wid], abv)

        def in_copy(ci, s):
            return pltpu.make_async_copy(
                state_hbm.at[pl.ds((bb + ci * CHUNK) * RPB, CHUNK * RPB), :],
                buf.at[s], sin_[s])

        def out_copy(ci, s):
            return pltpu.make_async_copy(
                buf.at[s],
                out_hbm.at[pl.ds((bb + ci * CHUNK) * RPB, CHUNK * RPB), :],
                sout[s])

        for s in range(NSLOT - 1):
            in_copy(s, s).start()

        def group_body(g, carry):
            for s in range(NSLOT):
                ci = NSLOT * g + s
                in_copy(ci, s).wait()
                for bi in range(CHUNK):
                    i = ci * CHUNK + bi
                    arow = abv[i // 8, pl.ds((i % 8) * L, L)]
                    brow = abv[4 + i // 8, pl.ds((i % 8) * L, L)]

                    def q_body(q, carry, arow=arow, brow=brow, bi=bi):
                        qv = jnp.full((L,), 0, jnp.int32) + q
                        aq = _permute(arow, qv)
                        bq = _permute(brow, qv)
                        r0 = bi * RPB + 2 * q
                        for t in range(8):
                            c = t * L
                            x0 = buf[s, r0, pl.ds(c, L)]
                            x1 = buf[s, r0 + 1, pl.ds(c, L)]
                            buf[s, r0, pl.ds(c, L)] = aq * x0 - bq * x1
                            buf[s, r0 + 1, pl.ds(c, L)] = aq * x1 - bq * x0
                        return carry

                    lax.fori_loop(0, Q, q_body, 0)
                out_copy(ci, s).start()
                s2 = (s + NSLOT - 1) % NSLOT
                nxt = ci + NSLOT - 1

                @pl.when(ci >= 1)
                def _():
                    out_copy(ci - 1, s2).wait()

                @pl.when(nxt < NCH)
                def _():
                    in_copy(nxt, s2).start()
            return carry

        lax.fori_loop(0, NCH // NSLOT, group_body, 0)
        out_copy(NCH - 1, (NCH - 1) % NSLOT).wait()

    return body(state2d, ab)


def kernel(state, weights):
    ab = _trig(weights.reshape(128, 128))
    s2 = state.transpose(0, 1, 3, 4, 2).reshape(B * RPB, 128)
    out = _sc_rotate(s2, ab)
    return out.reshape(B, Q, 2, 1, 128).transpose(0, 1, 4, 2, 3)


# speedup vs baseline: 1.0483x; 1.0483x over previous
import functools

import jax
import jax.numpy as jnp
from jax import lax
from jax.experimental import pallas as pl
from jax.experimental.pallas import tpu as pltpu
from jax.experimental.pallas import tpu_sc as plsc

B = 1024
Q = 16
ROW = Q * 128 * 2
NW = 32
BPW = B // NW
L = 16
RPB = ROW // 128
NSLOT = 8
CHUNK = 2
NCH = BPW // CHUNK

_PIB = lax.GatherScatterMode.PROMISE_IN_BOUNDS
_DNUMS = lax.GatherDimensionNumbers(
    offset_dims=(), collapsed_slice_dims=(0,), start_index_map=(0,))


def _permute(x, idx):
    return lax.gather(x, idx[:, None], _DNUMS, slice_sizes=(1,), mode=_PIB)


def _trig_body(w_ref, ab_ref):
    th = w_ref[...] * 0.5
    a = jnp.cos(th)
    b = jnp.sin(th)
    for w in range(NW):
        ab_ref[w, 0:4, :] = a[4 * w:4 * w + 4, :]
        ab_ref[w, 4:8, :] = b[4 * w:4 * w + 4, :]


def _trig(w2):
    return pl.pallas_call(
        _trig_body,
        out_shape=jax.ShapeDtypeStruct((NW, 8, 128), jnp.float32),
    )(w2)


def _sc_rotate(state2d, ab):
    mesh = plsc.VectorSubcoreMesh(core_axis_name="c", subcore_axis_name="s")

    @functools.partial(
        pl.kernel,
        mesh=mesh,
        out_type=jax.ShapeDtypeStruct((B * RPB, 128), jnp.float32),
        scratch_types=[
            pltpu.VMEM((NSLOT, CHUNK * RPB, 128), jnp.float32),
            pltpu.VMEM((8, 128), jnp.float32),
        ] + [pltpu.SemaphoreType.DMA] * (2 * NSLOT),
        compiler_params=pltpu.CompilerParams(use_tc_tiling_on_sc=True),
    )
    def body(state_hbm, ab_hbm, out_hbm, buf, abv, *sems):
        sin_, sout = sems[:NSLOT], sems[NSLOT:]
        wid = lax.axis_index("s") * 2 + lax.axis_index("c")
        bb = wid * BPW
        pltpu.sync_copy(ab_hbm.at[wid], abv)

        def in_copy(ci, s):
            return pltpu.make_async_copy(
                state_hbm.at[pl.ds((bb + ci * CHUNK) * RPB, CHUNK * RPB), :],
                buf.at[s], sin_[s])

        def out_copy(ci, s):
            return pltpu.make_async_copy(
                buf.at[s],
                out_hbm.at[pl.ds((bb + ci * CHUNK) * RPB, CHUNK * RPB), :],
                sout[s])

        for s in range(NSLOT - 1):
            in_copy(s, s).start()

        def group_body(g, carry):
            for s in range(NSLOT):
                ci = NSLOT * g + s
                in_copy(ci, s).wait()
                for bi in range(CHUNK):
                    i = ci * CHUNK + bi
                    arow = abv[i // 8, pl.ds((i % 8) * L, L)]
                    brow = abv[4 + i // 8, pl.ds((i % 8) * L, L)]

                    def q_body(q, carry, arow=arow, brow=brow, bi=bi):
                        qv = jnp.full((L,), 0, jnp.int32) + q
                        aq = _permute(arow, qv)
                        bq = _permute(brow, qv)
                        r0 = bi * RPB + 2 * q
                        for t in range(8):
                            c = t * L
                            x0 = buf[s, r0, pl.ds(c, L)]
                            x1 = buf[s, r0 + 1, pl.ds(c, L)]
                            buf[s, r0, pl.ds(c, L)] = aq * x0 - bq * x1
                            buf[s, r0 + 1, pl.ds(c, L)] = aq * x1 - bq * x0
                        return carry

                    lax.fori_loop(0, Q, q_body, 0)
                out_copy(ci, s).start()
                s2 = (s + NSLOT - 1) % NSLOT
                nxt = ci + NSLOT - 1

                @pl.when(ci >= 1)
                def _():
                    out_copy(ci - 1, s2).wait()

                @pl.when(nxt < NCH)
                def _():
                    in_copy(nxt, s2).start()
            return carry

        lax.fori_loop(0, NCH // NSLOT, group_body, 0)
        out_copy(NCH - 1, (NCH - 1) % NSLOT).wait()

    return body(state2d, ab)


def kernel(state, weights):
    ab = _trig(weights.reshape(128, 128))
    s2 = state.transpose(0, 1, 3, 4, 2).reshape(B * RPB, 128)
    out = _sc_rotate(s2, ab)
    return out.reshape(B, Q, 2, 1, 128).transpose(0, 1, 4, 2, 3)
